# baseline (speedup 1.0000x reference)
import functools

import jax
import jax.numpy as jnp
import numpy as np
from jax import lax
from jax.experimental import pallas as pl
from jax.experimental.pallas import tpu as pltpu
from jax.experimental.pallas import tpu_sc as plsc

S = 2048
D = 1024
H = 16
DH = 64
F = 2048
NL = 2
ROUNDS = 2
BL = 64
NC = S // BL
NB = 32
NP = ROUNDS * H
NCLS = 10
SB = 256
HIGH = lax.Precision.HIGHEST

_f32 = jnp.float32
_i32 = jnp.int32


def _dot(a, b):
    return lax.dot_general(a, b, (((1,), (0,)), ((), ())),
                           precision=lax.Precision.DEFAULT,
                           preferred_element_type=_f32)


def _dot_hi(a, b):
    return lax.dot_general(a, b, (((1,), (0,)), ((), ())), precision=HIGH,
                           preferred_element_type=_f32)


def _ln(x, g, b):
    m = jnp.mean(x, -1, keepdims=True)
    v = jnp.mean((x - m) * (x - m), -1, keepdims=True)
    return g * (x - m) / jnp.sqrt(v + 1e-5) + b


def _pre_body(x_ref, g_ref, b_ref, wpack_ref, out_ref):
    xn = _ln(x_ref[...], g_ref[...], b_ref[...])
    out_ref[...] = _dot(xn, wpack_ref[...])


def _pre(x2, g, b, Wpack):
    return pl.pallas_call(
        _pre_body,
        grid=(S // SB,),
        in_specs=[
            pl.BlockSpec((SB, D), lambda i: (i, 0)),
            pl.BlockSpec((1, D), lambda i: (0, 0)),
            pl.BlockSpec((1, D), lambda i: (0, 0)),
            pl.BlockSpec((D, 2 * D), lambda i: (0, 0)),
        ],
        out_specs=pl.BlockSpec((SB, 2 * D), lambda i: (i, 0)),
        out_shape=jax.ShapeDtypeStruct((S, 2 * D), _f32),
    )(x2, g, b, Wpack)


def _route_body(qk_ref, r_ref, uidx_ref, bkt_ref):
    qkh = qk_ref[:, 0, 0, :]
    rot = _dot(qkh, r_ref[0])
    rc = jnp.concatenate([rot, -rot], axis=1)
    mx = jnp.max(rc, axis=1, keepdims=True)
    io = lax.broadcasted_iota(_i32, (S, NB), 1)
    bkt = jnp.min(jnp.where(rc >= mx, io, 2 ** 30), axis=1, keepdims=True)
    oh = (bkt == io).astype(_f32)
    counts = jnp.sum(oh, axis=0, keepdims=True)
    upper = (lax.broadcasted_iota(_i32, (NB, NB), 0)
             < lax.broadcasted_iota(_i32, (NB, NB), 1)).astype(_f32)
    offs = _dot_hi(counts, upper)
    ltri = (lax.broadcasted_iota(_i32, (SB, SB), 0)
            > lax.broadcasted_iota(_i32, (SB, SB), 1)).astype(_f32)
    running = jnp.zeros((1, NB), _f32)
    for i in range(S // SB):
        ohb = oh[i * SB:(i + 1) * SB]
        excl = _dot_hi(ltri, ohb) + running
        rank = jnp.sum(excl * ohb, axis=1, keepdims=True)
        base = jnp.sum(offs * ohb, axis=1, keepdims=True)
        uidx_ref[0, i * SB:(i + 1) * SB, :] = (rank + base).astype(_i32)
        running = running + jnp.sum(ohb, axis=0, keepdims=True)
    bkt_ref[0] = bkt


def _route(qkv4, Rr):
    return pl.pallas_call(
        _route_body,
        grid=(NP,),
        in_specs=[
            pl.BlockSpec((S, 1, 2, DH), lambda p: (0, p % H, 0, 0)),
            pl.BlockSpec((1, DH, NB // 2), lambda p: (p // H, 0, 0)),
        ],
        out_specs=[
            pl.BlockSpec((1, S, 1), lambda p: (p, 0, 0)),
            pl.BlockSpec((1, S, 1), lambda p: (p, 0, 0)),
        ],
        out_shape=[
            jax.ShapeDtypeStruct((NP, S, 1), _i32),
            jax.ShapeDtypeStruct((NP, S, 1), _i32),
        ],
    )(qkv4, Rr)


def _sc_sort_gather_body(uidx_hbm, bkt_hbm, qkv_hbm,
                         sqkv_hbm, sb_hbm, sp_hbm,
                         uidx_v, bkt_v, sidx_v, sb_v, g_v, rb0, rb1,
                         sem0, sem1):
    p = lax.axis_index("s") * 2 + lax.axis_index("c")
    h = lax.rem(p, H)
    pltpu.sync_copy(uidx_hbm.at[p], uidx_v)
    pltpu.sync_copy(bkt_hbm.at[p], bkt_v)

    def inv_body(i, _):
        idx = uidx_v[pl.ds(i * 16, 16)]
        plsc.store_scatter(sidx_v, [idx], lax.iota(_i32, 16) + i * 16)
        return 0
    lax.fori_loop(0, S // 16, inv_body, 0)

    bufs = ((rb0, sem0), (rb1, sem1))
    prev = None
    for c in range(S // 128):
        for j in range(8):
            i = c * 8 + j
            sv = sidx_v[pl.ds(i * 16, 16)]
            sb_v[pl.ds(i * 16, 16)] = plsc.load_gather(bkt_v, [sv])
            g_v[pl.ds(i * 16, 16)] = sv * H + h
        buf, sem = bufs[c % 2]
        cp = pltpu.async_copy(qkv_hbm.at[g_v.at[pl.ds(c * 128, 128)]],
                              buf, sem)
        if prev is not None:
            pcp, pbuf, pc = prev
            pcp.wait()
            pltpu.sync_copy(pbuf, sqkv_hbm.at[p, pl.ds(pc * 128, 128)])
        prev = (cp, buf, c)
    pcp, pbuf, pc = prev
    pcp.wait()
    pltpu.sync_copy(pbuf, sqkv_hbm.at[p, pl.ds(pc * 128, 128)])
    pltpu.sync_copy(sb_v, sb_hbm.at[p])
    pltpu.sync_copy(sidx_v, sp_hbm.at[p])


@functools.cache
def _sc_sort_gather():
    mesh = plsc.VectorSubcoreMesh(core_axis_name="c", subcore_axis_name="s")
    return pl.kernel(
        _sc_sort_gather_body,
        out_type=[
            jax.ShapeDtypeStruct((NP, S, 2 * DH), _f32),
            jax.ShapeDtypeStruct((NP, S), _i32),
            jax.ShapeDtypeStruct((NP, S), _i32),
        ],
        mesh=mesh,
        scratch_types=[
            pltpu.VMEM((S,), _i32),
            pltpu.VMEM((S,), _i32),
            pltpu.VMEM((S,), _i32),
            pltpu.VMEM((S,), _i32),
            pltpu.VMEM((S,), _i32),
            pltpu.VMEM((128, 2 * DH), _f32),
            pltpu.VMEM((128, 2 * DH), _f32),
            pltpu.SemaphoreType.DMA,
            pltpu.SemaphoreType.DMA,
        ],
        compiler_params=pltpu.CompilerParams(needs_layout_passes=False),
    )


QB = 256
KW = QB + BL
NQB = S // QB


def _attn_body(qkv_ref, sbq_ref, sbk_ref, spq_ref, spk_ref, po_ref):
    qci = lax.broadcasted_iota(_i32, (QB, KW), 0) // BL
    kci = lax.broadcasted_iota(_i32, (QB, KW), 1) // BL
    band_pen = jnp.where((kci == qci) | (kci == qci + 1), 0.0, -1e9)

    def block(qb, _):
        cprev = lax.rem(qb * 4 + NC - 1, NC)
        qrows = qkv_ref[0, pl.ds(qb * QB, QB), :]
        krows = jnp.concatenate(
            [qkv_ref[0, pl.ds(cprev * BL, BL), :], qrows],
            axis=0)
        q = qrows[:, :DH]
        k = krows[:, :DH]
        nrm = jnp.sqrt(jnp.sum(k * k, axis=1, keepdims=True)) + 1e-6
        kn = k / nrm
        v2 = krows[:, DH:]
        scores = lax.dot_general(q, kn, (((1,), (1,)), ((), ())),
                                 precision=lax.Precision.DEFAULT,
                                 preferred_element_type=_f32)
        scores = scores * (1.0 / np.sqrt(DH)) + band_pen
        bq = sbq_ref[0, pl.ds(qb * QB, QB), :]
        pq = spq_ref[0, pl.ds(qb * QB, QB), :]
        bk = jnp.concatenate(
            [sbk_ref[0, pl.ds(cprev, 1), :]]
            + [sbk_ref[0, pl.ds(qb * 4 + j, 1), :] for j in range(4)],
            axis=1)
        pk = jnp.concatenate(
            [spk_ref[0, pl.ds(cprev, 1), :]]
            + [spk_ref[0, pl.ds(qb * 4 + j, 1), :] for j in range(4)],
            axis=1)
        scores = scores + jnp.where(bq == bk, 0.0, -1e9)
        scores = scores + jnp.where(pq == pk, -1e5, 0.0)
        m = jnp.max(scores, axis=1, keepdims=True)
        ex = jnp.exp(scores - m)
        sx = jnp.sum(ex, axis=1, keepdims=True)
        lse = m + jnp.log(sx)
        attn = ex / sx
        o = lax.dot_general(attn, v2, (((1,), (0,)), ((), ())),
                            precision=lax.Precision.DEFAULT,
                            preferred_element_type=_f32)
        packed = jnp.concatenate(
            [o, lse, jnp.zeros((QB, DH - 1), _f32)], axis=1)
        po_ref[0, pl.ds(qb * QB, QB), :] = packed
        return 0
    lax.fori_loop(0, NQB, block, 0)


def _attn(sqkv, sbq, sbk, spq, spk):
    return pl.pallas_call(
        _attn_body,
        grid=(NP,),
        in_specs=[
            pl.BlockSpec((1, S, 2 * DH), lambda p: (p, 0, 0)),
            pl.BlockSpec((1, S, 1), lambda p: (p, 0, 0)),
            pl.BlockSpec((1, NC, BL), lambda p: (p, 0, 0)),
            pl.BlockSpec((1, S, 1), lambda p: (p, 0, 0)),
            pl.BlockSpec((1, NC, BL), lambda p: (p, 0, 0)),
        ],
        out_specs=pl.BlockSpec((1, S, 2 * DH), lambda p: (p, 0, 0)),
        out_shape=jax.ShapeDtypeStruct((NP, S, 2 * DH), _f32),
    )(sqkv, sbq, sbk, spq, spk)


def _sc_unsort_body(uidx_hbm, of_hbm, ou_hbm, uidx_v, g_v, rb0, rb1,
                    sem0, sem1):
    p = lax.axis_index("s") * 2 + lax.axis_index("c")
    pltpu.sync_copy(uidx_hbm.at[p], uidx_v)

    def idx_body(i, _):
        g_v[pl.ds(i * 16, 16)] = uidx_v[pl.ds(i * 16, 16)] + p * S
        return 0
    lax.fori_loop(0, S // 16, idx_body, 0)

    bufs = ((rb0, sem0), (rb1, sem1))
    prev = None
    for c in range(S // 128):
        buf, sem = bufs[c % 2]
        cp = pltpu.async_copy(of_hbm.at[g_v.at[pl.ds(c * 128, 128)]],
                              buf, sem)
        if prev is not None:
            pcp, pbuf, pc = prev
            pcp.wait()
            pltpu.sync_copy(pbuf, ou_hbm.at[p, pl.ds(pc * 128, 128)])
        prev = (cp, buf, c)
    pcp, pbuf, pc = prev
    pcp.wait()
    pltpu.sync_copy(pbuf, ou_hbm.at[p, pl.ds(pc * 128, 128)])


@functools.cache
def _sc_unsort():
    mesh = plsc.VectorSubcoreMesh(core_axis_name="c", subcore_axis_name="s")
    return pl.kernel(
        _sc_unsort_body,
        out_type=[
            jax.ShapeDtypeStruct((NP, S, 2 * DH), _f32),
        ],
        mesh=mesh,
        scratch_types=[
            pltpu.VMEM((S,), _i32),
            pltpu.VMEM((S,), _i32),
            pltpu.VMEM((128, 2 * DH), _f32),
            pltpu.VMEM((128, 2 * DH), _f32),
            pltpu.SemaphoreType.DMA,
            pltpu.SemaphoreType.DMA,
        ],
        compiler_params=pltpu.CompilerParams(needs_layout_passes=False),
    )


def _comb_body(o_ref, wo_ref, x1_ref, out_ref):
    hh = pl.program_id(0)
    p0 = o_ref[0, 0]
    p1 = o_ref[1, 0]
    l0 = p0[:, DH:DH + 1]
    l1 = p1[:, DH:DH + 1]
    m = jnp.maximum(l0, l1)
    e0 = jnp.exp(l0 - m)
    e1 = jnp.exp(l1 - m)
    inv = 1.0 / (e0 + e1)
    comb = (e0 * inv) * p0[:, :DH] + (e1 * inv) * p1[:, :DH]
    contrib = _dot(comb, wo_ref[...])

    @pl.when(hh == 0)
    def _():
        out_ref[...] = x1_ref[...]

    out_ref[...] += contrib


def _comb(o_u, Wo, x1):
    return pl.pallas_call(
        _comb_body,
        grid=(H,),
        in_specs=[
            pl.BlockSpec((ROUNDS, 1, S, 2 * DH), lambda h: (0, h, 0, 0)),
            pl.BlockSpec((DH, D), lambda h: (h, 0)),
            pl.BlockSpec((S, D), lambda h: (0, 0)),
        ],
        out_specs=pl.BlockSpec((S, D), lambda h: (0, 0)),
        out_shape=jax.ShapeDtypeStruct((S, D), _f32),
    )(o_u, Wo, x1)


def _ffn_body(y1_ref, x2_ref, g_ref, b_ref, w1_ref, b1_ref, w2_ref, b2_ref,
              out_ref):
    hn = _ln(y1_ref[...], g_ref[...], b_ref[...])
    t = jnp.maximum(_dot(hn, w1_ref[...]) + b1_ref[...], 0.0)
    out_ref[...] = x2_ref[...] + _dot(t, w2_ref[...]) + b2_ref[...]


def _ffn(y1, x2, g, b, W1, b1, W2, b2):
    return pl.pallas_call(
        _ffn_body,
        grid=(S // SB,),
        in_specs=[
            pl.BlockSpec((SB, D), lambda i: (i, 0)),
            pl.BlockSpec((SB, D), lambda i: (i, 0)),
            pl.BlockSpec((1, D), lambda i: (0, 0)),
            pl.BlockSpec((1, D), lambda i: (0, 0)),
            pl.BlockSpec((D, F), lambda i: (0, 0)),
            pl.BlockSpec((1, F), lambda i: (0, 0)),
            pl.BlockSpec((F, D), lambda i: (0, 0)),
            pl.BlockSpec((1, D), lambda i: (0, 0)),
        ],
        out_specs=pl.BlockSpec((SB, D), lambda i: (i, 0)),
        out_shape=jax.ShapeDtypeStruct((S, D), _f32),
    )(y1, x2, g, b, W1, b1, W2, b2)


def _cls_body(x_ref, w1_ref, b1_ref, w2_ref, b2_ref, out_ref):
    t = jnp.maximum(_dot(x_ref[...], w1_ref[...]) + b1_ref[...], 0.0)
    out_ref[...] = _dot(t, w2_ref[...]) + b2_ref[...]


def _cls(x2, Wc1, bc1, Wc2p, bc2p):
    return pl.pallas_call(
        _cls_body,
        grid=(S // SB,),
        in_specs=[
            pl.BlockSpec((SB, D), lambda i: (i, 0)),
            pl.BlockSpec((D, 2 * D), lambda i: (0, 0)),
            pl.BlockSpec((1, 2 * D), lambda i: (0, 0)),
            pl.BlockSpec((2 * D, 128), lambda i: (0, 0)),
            pl.BlockSpec((1, 128), lambda i: (0, 0)),
        ],
        out_specs=pl.BlockSpec((SB, 128), lambda i: (i, 0)),
        out_shape=jax.ShapeDtypeStruct((S, 128), _f32),
    )(x2, Wc1, bc1, Wc2p, bc2p)


def kernel(inputs, Wqk, Wv, Wo, ln1_g, ln1_b, ln2_g, ln2_b, W1, b1, W2, b2,
           Wc1, bc1, Wc2, bc2):
    x0 = inputs.reshape(S, D)
    x1 = x0
    x2 = x0
    for li in range(NL):
        rkey = jax.random.fold_in(jax.random.key(123), li)
        R = jax.random.normal(rkey, (DH, ROUNDS, NB // 2), dtype=_f32)
        Rr = R.transpose(1, 0, 2)
        Wpack = jnp.concatenate(
            [Wqk[li].reshape(D, H, 1, DH), Wv[li].reshape(D, H, 1, DH)],
            axis=2).reshape(D, 2 * D)
        qkv = _pre(x2, ln1_g[li].reshape(1, D), ln1_b[li].reshape(1, D),
                   Wpack)
        uidx3, bkt3 = _route(qkv.reshape(S, H, 2, DH), Rr)
        uidx = uidx3.reshape(NP, S)
        bkt = bkt3.reshape(NP, S)
        sqkv, sb, sp = _sc_sort_gather()(uidx, bkt,
                                         qkv.reshape(S * H, 2 * DH))
        sbf = sb.astype(_f32)
        spf = sp.astype(_f32)
        po = _attn(sqkv,
                   sbf.reshape(NP, S, 1), sbf.reshape(NP, NC, BL),
                   spf.reshape(NP, S, 1), spf.reshape(NP, NC, BL))
        (o_u,) = _sc_unsort()(uidx, po.reshape(NP * S, 2 * DH))
        y1 = _comb(o_u.reshape(ROUNDS, H, S, 2 * DH), Wo[li], x1)
        y2 = _ffn(y1, x2, ln2_g[li].reshape(1, D), ln2_b[li].reshape(1, D),
                  W1[li], b1[li].reshape(1, F), W2[li], b2[li].reshape(1, D))
        x1, x2 = y1, y2
    Wc2p = jnp.pad(Wc2, ((0, 0), (0, 128 - NCLS)))
    bc2p = jnp.pad(bc2, (0, 128 - NCLS)).reshape(1, 128)
    logits = _cls(x2, Wc1, bc1.reshape(1, 2 * D), Wc2p, bc2p)
    return logits[:, :NCLS].reshape(1, S, NCLS)


# speedup vs baseline: 1.0882x; 1.0882x over previous
import functools

import jax
import jax.numpy as jnp
import numpy as np
from jax import lax
from jax.experimental import pallas as pl
from jax.experimental.pallas import tpu as pltpu
from jax.experimental.pallas import tpu_sc as plsc

S = 2048
D = 1024
H = 16
DH = 64
F = 2048
NL = 2
ROUNDS = 2
BL = 64
NC = S // BL
NB = 32
NP = ROUNDS * H
NCLS = 10
SB = 256
HIGH = lax.Precision.HIGHEST

_f32 = jnp.float32
_i32 = jnp.int32


def _dot(a, b):
    return lax.dot_general(a, b, (((1,), (0,)), ((), ())),
                           precision=lax.Precision.DEFAULT,
                           preferred_element_type=_f32)


def _dot_hi(a, b):
    return lax.dot_general(a, b, (((1,), (0,)), ((), ())), precision=HIGH,
                           preferred_element_type=_f32)


def _ln(x, g, b):
    m = jnp.mean(x, -1, keepdims=True)
    v = jnp.mean((x - m) * (x - m), -1, keepdims=True)
    return g * (x - m) / jnp.sqrt(v + 1e-5) + b


def _pre_body(x_ref, g_ref, b_ref, wpack_ref, rcat_ref, out_ref, rot_ref):
    xn = _ln(x_ref[...], g_ref[...], b_ref[...])
    out = _dot(xn, wpack_ref[...])
    out_ref[...] = out
    rc = rcat_ref[...]
    for p in range(NP):
        h, r = p % H, p // H
        rot_ref[p] = _dot(out[:, h * 2 * DH:h * 2 * DH + DH],
                          rc[:, r * 16:r * 16 + 16])


def _pre(x2, g, b, Wpack, Rcat):
    return pl.pallas_call(
        _pre_body,
        grid=(S // SB,),
        in_specs=[
            pl.BlockSpec((SB, D), lambda i: (i, 0)),
            pl.BlockSpec((1, D), lambda i: (0, 0)),
            pl.BlockSpec((1, D), lambda i: (0, 0)),
            pl.BlockSpec((D, 2 * D), lambda i: (0, 0)),
            pl.BlockSpec((DH, ROUNDS * 16), lambda i: (0, 0)),
        ],
        out_specs=[
            pl.BlockSpec((SB, 2 * D), lambda i: (i, 0)),
            pl.BlockSpec((NP, SB, 16), lambda i: (0, i, 0)),
        ],
        out_shape=[
            jax.ShapeDtypeStruct((S, 2 * D), _f32),
            jax.ShapeDtypeStruct((NP, S, 16), _f32),
        ],
    )(x2, g, b, Wpack, Rcat)


def _route_body(rot_ref, uidx_ref, bkt_ref):
    io = lax.broadcasted_iota(_i32, (S, NB), 1)
    upper = (lax.broadcasted_iota(_i32, (NB, NB), 0)
             < lax.broadcasted_iota(_i32, (NB, NB), 1)).astype(_f32)
    ltri = (lax.broadcasted_iota(_i32, (SB, SB), 0)
            > lax.broadcasted_iota(_i32, (SB, SB), 1)).astype(_f32)
    rh = rot_ref[0]
    rc = jnp.concatenate([rh, -rh], axis=1)
    mx = jnp.max(rc, axis=1, keepdims=True)
    bkt = jnp.min(jnp.where(rc >= mx, io, 2 ** 30), axis=1, keepdims=True)
    oh = (bkt == io).astype(_f32)
    counts = jnp.sum(oh, axis=0, keepdims=True)
    offs = _dot_hi(counts, upper)
    running = jnp.zeros((1, NB), _f32)
    for i in range(S // SB):
        ohb = oh[i * SB:(i + 1) * SB]
        excl = _dot_hi(ltri, ohb) + running
        rank = jnp.sum(excl * ohb, axis=1, keepdims=True)
        base = jnp.sum(offs * ohb, axis=1, keepdims=True)
        uidx_ref[0, i * SB:(i + 1) * SB, :] = (rank + base).astype(_i32)
        running = running + jnp.sum(ohb, axis=0, keepdims=True)
    bkt_ref[0] = bkt


def _route(rot):
    return pl.pallas_call(
        _route_body,
        grid=(NP,),
        in_specs=[
            pl.BlockSpec((1, S, 16), lambda p: (p, 0, 0)),
        ],
        out_specs=[
            pl.BlockSpec((1, S, 1), lambda p: (p, 0, 0)),
            pl.BlockSpec((1, S, 1), lambda p: (p, 0, 0)),
        ],
        out_shape=[
            jax.ShapeDtypeStruct((NP, S, 1), _i32),
            jax.ShapeDtypeStruct((NP, S, 1), _i32),
        ],
    )(rot)


def _sc_sort_gather_body(uidx_hbm, bkt_hbm, qkv_hbm,
                         sqkv_hbm, sb_hbm, sp_hbm,
                         uidx_v, bkt_v, sidx_v, sb_v, g_v, rb0, rb1,
                         sem0, sem1):
    p = lax.axis_index("s") * 2 + lax.axis_index("c")
    h = lax.rem(p, H)
    pltpu.sync_copy(uidx_hbm.at[p], uidx_v)
    pltpu.sync_copy(bkt_hbm.at[p], bkt_v)

    def inv_body(i, _):
        idx = uidx_v[pl.ds(i * 16, 16)]
        plsc.store_scatter(sidx_v, [idx], lax.iota(_i32, 16) + i * 16)
        return 0
    lax.fori_loop(0, S // 16, inv_body, 0)

    bufs = ((rb0, sem0), (rb1, sem1))
    prev = None
    for c in range(S // 128):
        for j in range(8):
            i = c * 8 + j
            sv = sidx_v[pl.ds(i * 16, 16)]
            sb_v[pl.ds(i * 16, 16)] = plsc.load_gather(bkt_v, [sv])
            g_v[pl.ds(i * 16, 16)] = sv * H + h
        buf, sem = bufs[c % 2]
        cp = pltpu.async_copy(qkv_hbm.at[g_v.at[pl.ds(c * 128, 128)]],
                              buf, sem)
        if prev is not None:
            pcp, pbuf, pc = prev
            pcp.wait()
            pltpu.sync_copy(pbuf, sqkv_hbm.at[p, pl.ds(pc * 128, 128)])
        prev = (cp, buf, c)
    pcp, pbuf, pc = prev
    pcp.wait()
    pltpu.sync_copy(pbuf, sqkv_hbm.at[p, pl.ds(pc * 128, 128)])
    pltpu.sync_copy(sb_v, sb_hbm.at[p])
    pltpu.sync_copy(sidx_v, sp_hbm.at[p])


@functools.cache
def _sc_sort_gather():
    mesh = plsc.VectorSubcoreMesh(core_axis_name="c", subcore_axis_name="s")
    return pl.kernel(
        _sc_sort_gather_body,
        out_type=[
            jax.ShapeDtypeStruct((NP, S, 2 * DH), _f32),
            jax.ShapeDtypeStruct((NP, S), _i32),
            jax.ShapeDtypeStruct((NP, S), _i32),
        ],
        mesh=mesh,
        scratch_types=[
            pltpu.VMEM((S,), _i32),
            pltpu.VMEM((S,), _i32),
            pltpu.VMEM((S,), _i32),
            pltpu.VMEM((S,), _i32),
            pltpu.VMEM((S,), _i32),
            pltpu.VMEM((128, 2 * DH), _f32),
            pltpu.VMEM((128, 2 * DH), _f32),
            pltpu.SemaphoreType.DMA,
            pltpu.SemaphoreType.DMA,
        ],
        compiler_params=pltpu.CompilerParams(needs_layout_passes=False),
    )


QB = 256
KW = QB + BL
NQB = S // QB


def _attn_body(qkv_ref, sbq_ref, sbk_ref, spq_ref, spk_ref, po_ref):
    qci = lax.broadcasted_iota(_i32, (QB, KW), 0) // BL
    kci = lax.broadcasted_iota(_i32, (QB, KW), 1) // BL
    band_pen = jnp.where((kci == qci) | (kci == qci + 1), 0.0, -1e9)

    def block(qb, _):
        cprev = lax.rem(qb * 4 + NC - 1, NC)
        qrows = qkv_ref[0, pl.ds(qb * QB, QB), :]
        krows = jnp.concatenate(
            [qkv_ref[0, pl.ds(cprev * BL, BL), :], qrows],
            axis=0)
        q = qrows[:, :DH]
        k = krows[:, :DH]
        nrm = jnp.sqrt(jnp.sum(k * k, axis=1, keepdims=True)) + 1e-6
        kn = k / nrm
        v2 = krows[:, DH:]
        scores = lax.dot_general(q, kn, (((1,), (1,)), ((), ())),
                                 precision=lax.Precision.DEFAULT,
                                 preferred_element_type=_f32)
        scores = scores * (1.0 / np.sqrt(DH)) + band_pen
        bq = sbq_ref[0, pl.ds(qb * QB, QB), :]
        pq = spq_ref[0, pl.ds(qb * QB, QB), :]
        bk = jnp.concatenate(
            [sbk_ref[0, pl.ds(cprev, 1), :]]
            + [sbk_ref[0, pl.ds(qb * 4 + j, 1), :] for j in range(4)],
            axis=1)
        pk = jnp.concatenate(
            [spk_ref[0, pl.ds(cprev, 1), :]]
            + [spk_ref[0, pl.ds(qb * 4 + j, 1), :] for j in range(4)],
            axis=1)
        scores = scores + jnp.where(bq == bk, 0.0, -1e9)
        scores = scores + jnp.where(pq == pk, -1e5, 0.0)
        m = jnp.max(scores, axis=1, keepdims=True)
        ex = jnp.exp(scores - m)
        sx = jnp.sum(ex, axis=1, keepdims=True)
        lse = m + jnp.log(sx)
        attn = ex / sx
        o = lax.dot_general(attn, v2, (((1,), (0,)), ((), ())),
                            precision=lax.Precision.DEFAULT,
                            preferred_element_type=_f32)
        packed = jnp.concatenate(
            [o, lse, jnp.zeros((QB, DH - 1), _f32)], axis=1)
        po_ref[0, pl.ds(qb * QB, QB), :] = packed
        return 0
    lax.fori_loop(0, NQB, block, 0)


def _attn(sqkv, sbq, sbk, spq, spk):
    return pl.pallas_call(
        _attn_body,
        grid=(NP,),
        in_specs=[
            pl.BlockSpec((1, S, 2 * DH), lambda p: (p, 0, 0)),
            pl.BlockSpec((1, S, 1), lambda p: (p, 0, 0)),
            pl.BlockSpec((1, NC, BL), lambda p: (p, 0, 0)),
            pl.BlockSpec((1, S, 1), lambda p: (p, 0, 0)),
            pl.BlockSpec((1, NC, BL), lambda p: (p, 0, 0)),
        ],
        out_specs=pl.BlockSpec((1, S, 2 * DH), lambda p: (p, 0, 0)),
        out_shape=jax.ShapeDtypeStruct((NP, S, 2 * DH), _f32),
    )(sqkv, sbq, sbk, spq, spk)


def _sc_unsort_body(uidx_hbm, of_hbm, ou_hbm, uidx_v, g_v, rb0, rb1,
                    sem0, sem1):
    p = lax.axis_index("s") * 2 + lax.axis_index("c")
    pltpu.sync_copy(uidx_hbm.at[p], uidx_v)

    def idx_body(i, _):
        g_v[pl.ds(i * 16, 16)] = uidx_v[pl.ds(i * 16, 16)] + p * S
        return 0
    lax.fori_loop(0, S // 16, idx_body, 0)

    bufs = ((rb0, sem0), (rb1, sem1))
    prev = None
    for c in range(S // 128):
        buf, sem = bufs[c % 2]
        cp = pltpu.async_copy(of_hbm.at[g_v.at[pl.ds(c * 128, 128)]],
                              buf, sem)
        if prev is not None:
            pcp, pbuf, pc = prev
            pcp.wait()
            pltpu.sync_copy(pbuf, ou_hbm.at[p, pl.ds(pc * 128, 128)])
        prev = (cp, buf, c)
    pcp, pbuf, pc = prev
    pcp.wait()
    pltpu.sync_copy(pbuf, ou_hbm.at[p, pl.ds(pc * 128, 128)])


@functools.cache
def _sc_unsort():
    mesh = plsc.VectorSubcoreMesh(core_axis_name="c", subcore_axis_name="s")
    return pl.kernel(
        _sc_unsort_body,
        out_type=[
            jax.ShapeDtypeStruct((NP, S, 2 * DH), _f32),
        ],
        mesh=mesh,
        scratch_types=[
            pltpu.VMEM((S,), _i32),
            pltpu.VMEM((S,), _i32),
            pltpu.VMEM((128, 2 * DH), _f32),
            pltpu.VMEM((128, 2 * DH), _f32),
            pltpu.SemaphoreType.DMA,
            pltpu.SemaphoreType.DMA,
        ],
        compiler_params=pltpu.CompilerParams(needs_layout_passes=False),
    )


def _comb_body(o_ref, wo_ref, x1_ref, out_ref):
    hh = pl.program_id(0)
    p0 = o_ref[0, 0]
    p1 = o_ref[1, 0]
    l0 = p0[:, DH:DH + 1]
    l1 = p1[:, DH:DH + 1]
    m = jnp.maximum(l0, l1)
    e0 = jnp.exp(l0 - m)
    e1 = jnp.exp(l1 - m)
    inv = 1.0 / (e0 + e1)
    comb = (e0 * inv) * p0[:, :DH] + (e1 * inv) * p1[:, :DH]
    contrib = _dot(comb, wo_ref[...])

    @pl.when(hh == 0)
    def _():
        out_ref[...] = x1_ref[...]

    out_ref[...] += contrib


def _comb(o_u, Wo, x1):
    return pl.pallas_call(
        _comb_body,
        grid=(H,),
        in_specs=[
            pl.BlockSpec((ROUNDS, 1, S, 2 * DH), lambda h: (0, h, 0, 0)),
            pl.BlockSpec((DH, D), lambda h: (h, 0)),
            pl.BlockSpec((S, D), lambda h: (0, 0)),
        ],
        out_specs=pl.BlockSpec((S, D), lambda h: (0, 0)),
        out_shape=jax.ShapeDtypeStruct((S, D), _f32),
    )(o_u, Wo, x1)


def _ffn_body(y1_ref, x2_ref, g_ref, b_ref, w1_ref, b1_ref, w2_ref, b2_ref,
              out_ref):
    hn = _ln(y1_ref[...], g_ref[...], b_ref[...])
    t = jnp.maximum(_dot(hn, w1_ref[...]) + b1_ref[...], 0.0)
    out_ref[...] = x2_ref[...] + _dot(t, w2_ref[...]) + b2_ref[...]


def _ffn(y1, x2, g, b, W1, b1, W2, b2):
    return pl.pallas_call(
        _ffn_body,
        grid=(S // SB,),
        in_specs=[
            pl.BlockSpec((SB, D), lambda i: (i, 0)),
            pl.BlockSpec((SB, D), lambda i: (i, 0)),
            pl.BlockSpec((1, D), lambda i: (0, 0)),
            pl.BlockSpec((1, D), lambda i: (0, 0)),
            pl.BlockSpec((D, F), lambda i: (0, 0)),
            pl.BlockSpec((1, F), lambda i: (0, 0)),
            pl.BlockSpec((F, D), lambda i: (0, 0)),
            pl.BlockSpec((1, D), lambda i: (0, 0)),
        ],
        out_specs=pl.BlockSpec((SB, D), lambda i: (i, 0)),
        out_shape=jax.ShapeDtypeStruct((S, D), _f32),
    )(y1, x2, g, b, W1, b1, W2, b2)


def _cls_body(x_ref, w1_ref, b1_ref, w2_ref, b2_ref, out_ref):
    t = jnp.maximum(_dot(x_ref[...], w1_ref[...]) + b1_ref[...], 0.0)
    out_ref[...] = _dot(t, w2_ref[...]) + b2_ref[...]


def _cls(x2, Wc1, bc1, Wc2p, bc2p):
    return pl.pallas_call(
        _cls_body,
        grid=(S // SB,),
        in_specs=[
            pl.BlockSpec((SB, D), lambda i: (i, 0)),
            pl.BlockSpec((D, 2 * D), lambda i: (0, 0)),
            pl.BlockSpec((1, 2 * D), lambda i: (0, 0)),
            pl.BlockSpec((2 * D, 128), lambda i: (0, 0)),
            pl.BlockSpec((1, 128), lambda i: (0, 0)),
        ],
        out_specs=pl.BlockSpec((SB, 128), lambda i: (i, 0)),
        out_shape=jax.ShapeDtypeStruct((S, 128), _f32),
    )(x2, Wc1, bc1, Wc2p, bc2p)


def kernel(inputs, Wqk, Wv, Wo, ln1_g, ln1_b, ln2_g, ln2_b, W1, b1, W2, b2,
           Wc1, bc1, Wc2, bc2):
    x0 = inputs.reshape(S, D)
    x1 = x0
    x2 = x0
    for li in range(NL):
        rkey = jax.random.fold_in(jax.random.key(123), li)
        R = jax.random.normal(rkey, (DH, ROUNDS, NB // 2), dtype=_f32)
        Wpack = jnp.concatenate(
            [Wqk[li].reshape(D, H, 1, DH), Wv[li].reshape(D, H, 1, DH)],
            axis=2).reshape(D, 2 * D)
        qkv, rot = _pre(x2, ln1_g[li].reshape(1, D), ln1_b[li].reshape(1, D),
                        Wpack, R.reshape(DH, ROUNDS * 16))
        uidx3, bkt3 = _route(rot)
        uidx = uidx3.reshape(NP, S)
        bkt = bkt3.reshape(NP, S)
        sqkv, sb, sp = _sc_sort_gather()(uidx, bkt,
                                         qkv.reshape(S * H, 2 * DH))
        sbf = sb.astype(_f32)
        spf = sp.astype(_f32)
        po = _attn(sqkv,
                   sbf.reshape(NP, S, 1), sbf.reshape(NP, NC, BL),
                   spf.reshape(NP, S, 1), spf.reshape(NP, NC, BL))
        (o_u,) = _sc_unsort()(uidx, po.reshape(NP * S, 2 * DH))
        y1 = _comb(o_u.reshape(ROUNDS, H, S, 2 * DH), Wo[li], x1)
        y2 = _ffn(y1, x2, ln2_g[li].reshape(1, D), ln2_b[li].reshape(1, D),
                  W1[li], b1[li].reshape(1, F), W2[li], b2[li].reshape(1, D))
        x1, x2 = y1, y2
    Wc2p = jnp.pad(Wc2, ((0, 0), (0, 128 - NCLS)))
    bc2p = jnp.pad(bc2, (0, 128 - NCLS)).reshape(1, 128)
    logits = _cls(x2, Wc1, bc1.reshape(1, 2 * D), Wc2p, bc2p)
    return logits[:, :NCLS].reshape(1, S, NCLS)


# speedup vs baseline: 1.1252x; 1.0340x over previous
import functools

import jax
import jax.numpy as jnp
import numpy as np
from jax import lax
from jax.experimental import pallas as pl
from jax.experimental.pallas import tpu as pltpu
from jax.experimental.pallas import tpu_sc as plsc

S = 2048
D = 1024
H = 16
DH = 64
F = 2048
NL = 2
ROUNDS = 2
BL = 64
NC = S // BL
NB = 32
NP = ROUNDS * H
NCLS = 10
SB = 256
HIGH = lax.Precision.HIGHEST

_f32 = jnp.float32
_i32 = jnp.int32


def _dot(a, b):
    return lax.dot_general(a, b, (((1,), (0,)), ((), ())),
                           precision=lax.Precision.DEFAULT,
                           preferred_element_type=_f32)


def _dot_hi(a, b):
    return lax.dot_general(a, b, (((1,), (0,)), ((), ())), precision=HIGH,
                           preferred_element_type=_f32)


def _ln(x, g, b):
    m = jnp.mean(x, -1, keepdims=True)
    v = jnp.mean((x - m) * (x - m), -1, keepdims=True)
    return g * (x - m) / jnp.sqrt(v + 1e-5) + b


def _pre_body(x_ref, g_ref, b_ref, wpack_ref, rcat_ref, out_ref, rot_ref):
    xn = _ln(x_ref[...], g_ref[...], b_ref[...])
    out = _dot(xn, wpack_ref[...])
    out_ref[...] = out
    rc = rcat_ref[...]
    for p in range(NP):
        h, r = p % H, p // H
        rot_ref[p] = _dot(out[:, h * 2 * DH:h * 2 * DH + DH],
                          rc[:, r * 16:r * 16 + 16])


def _pre(x2, g, b, Wpack, Rcat):
    return pl.pallas_call(
        _pre_body,
        grid=(S // SB,),
        in_specs=[
            pl.BlockSpec((SB, D), lambda i: (i, 0)),
            pl.BlockSpec((1, D), lambda i: (0, 0)),
            pl.BlockSpec((1, D), lambda i: (0, 0)),
            pl.BlockSpec((D, 2 * D), lambda i: (0, 0)),
            pl.BlockSpec((DH, ROUNDS * 16), lambda i: (0, 0)),
        ],
        out_specs=[
            pl.BlockSpec((SB, 2 * D), lambda i: (i, 0)),
            pl.BlockSpec((NP, SB, 16), lambda i: (0, i, 0)),
        ],
        out_shape=[
            jax.ShapeDtypeStruct((S, 2 * D), _f32),
            jax.ShapeDtypeStruct((NP, S, 16), _f32),
        ],
    )(x2, g, b, Wpack, Rcat)


def _route_body(rot_ref, uidx_ref, bkt_ref):
    io = lax.broadcasted_iota(_i32, (S, NB), 1)
    upper = (lax.broadcasted_iota(_i32, (NB, NB), 0)
             < lax.broadcasted_iota(_i32, (NB, NB), 1)).astype(_f32)
    ltri = (lax.broadcasted_iota(_i32, (SB, SB), 0)
            > lax.broadcasted_iota(_i32, (SB, SB), 1)).astype(_f32)
    rh = rot_ref[0]
    rc = jnp.concatenate([rh, -rh], axis=1)
    mx = jnp.max(rc, axis=1, keepdims=True)
    bkt = jnp.min(jnp.where(rc >= mx, io, 2 ** 30), axis=1, keepdims=True)
    oh = (bkt == io).astype(_f32)
    counts = jnp.sum(oh, axis=0, keepdims=True)
    offs = _dot_hi(counts, upper)
    running = jnp.zeros((1, NB), _f32)
    for i in range(S // SB):
        ohb = oh[i * SB:(i + 1) * SB]
        excl = _dot_hi(ltri, ohb) + running
        rank = jnp.sum(excl * ohb, axis=1, keepdims=True)
        base = jnp.sum(offs * ohb, axis=1, keepdims=True)
        uidx_ref[0, i * SB:(i + 1) * SB, :] = (rank + base).astype(_i32)
        running = running + jnp.sum(ohb, axis=0, keepdims=True)
    bkt_ref[0] = bkt


def _route(rot):
    return pl.pallas_call(
        _route_body,
        grid=(NP,),
        in_specs=[
            pl.BlockSpec((1, S, 16), lambda p: (p, 0, 0)),
        ],
        out_specs=[
            pl.BlockSpec((1, S, 1), lambda p: (p, 0, 0)),
            pl.BlockSpec((1, S, 1), lambda p: (p, 0, 0)),
        ],
        out_shape=[
            jax.ShapeDtypeStruct((NP, S, 1), _i32),
            jax.ShapeDtypeStruct((NP, S, 1), _i32),
        ],
    )(rot)


def _sc_sort_gather_body(uidx_hbm, bkt_hbm, qkv_hbm,
                         sqkv_hbm, sb_hbm, sp_hbm,
                         uidx_v, bkt_v, sidx_v, sb_v, g_v, rb0, rb1,
                         sem0, sem1):
    p = lax.axis_index("s") * 2 + lax.axis_index("c")
    h = lax.rem(p, H)
    pltpu.sync_copy(uidx_hbm.at[p], uidx_v)
    pltpu.sync_copy(bkt_hbm.at[p], bkt_v)

    def inv_body(i, _):
        idx = uidx_v[pl.ds(i * 16, 16)]
        plsc.store_scatter(sidx_v, [idx], lax.iota(_i32, 16) + i * 16)
        return 0
    lax.fori_loop(0, S // 16, inv_body, 0)

    bufs = ((rb0, sem0), (rb1, sem1))
    prev = None
    for c in range(S // 128):
        for j in range(8):
            i = c * 8 + j
            sv = sidx_v[pl.ds(i * 16, 16)]
            sb_v[pl.ds(i * 16, 16)] = plsc.load_gather(bkt_v, [sv])
            g_v[pl.ds(i * 16, 16)] = sv * H + h
        buf, sem = bufs[c % 2]
        cp = pltpu.async_copy(qkv_hbm.at[g_v.at[pl.ds(c * 128, 128)]],
                              buf, sem)
        if prev is not None:
            pcp, pbuf, pc = prev
            pcp.wait()
            pltpu.sync_copy(pbuf, sqkv_hbm.at[p, pl.ds(pc * 128, 128)])
        prev = (cp, buf, c)
    pcp, pbuf, pc = prev
    pcp.wait()
    pltpu.sync_copy(pbuf, sqkv_hbm.at[p, pl.ds(pc * 128, 128)])
    pltpu.sync_copy(sb_v, sb_hbm.at[p])
    pltpu.sync_copy(sidx_v, sp_hbm.at[p])


@functools.cache
def _sc_sort_gather():
    mesh = plsc.VectorSubcoreMesh(core_axis_name="c", subcore_axis_name="s")
    return pl.kernel(
        _sc_sort_gather_body,
        out_type=[
            jax.ShapeDtypeStruct((NP, S, 2 * DH), _f32),
            jax.ShapeDtypeStruct((NP, S), _i32),
            jax.ShapeDtypeStruct((NP, S), _i32),
        ],
        mesh=mesh,
        scratch_types=[
            pltpu.VMEM((S,), _i32),
            pltpu.VMEM((S,), _i32),
            pltpu.VMEM((S,), _i32),
            pltpu.VMEM((S,), _i32),
            pltpu.VMEM((S,), _i32),
            pltpu.VMEM((128, 2 * DH), _f32),
            pltpu.VMEM((128, 2 * DH), _f32),
            pltpu.SemaphoreType.DMA,
            pltpu.SemaphoreType.DMA,
        ],
        compiler_params=pltpu.CompilerParams(needs_layout_passes=False),
    )


QB = 256
KW = QB + BL
NQB = S // QB


def _attn_body(qkv_ref, sbq_ref, sbk_ref, spq_ref, spk_ref, po_ref):
    qci = lax.broadcasted_iota(_i32, (QB, KW), 0) // BL
    kci = lax.broadcasted_iota(_i32, (QB, KW), 1) // BL
    band_pen = jnp.where((kci == qci) | (kci == qci + 1), 0.0, -1e9)

    def block(qb, _):
        cprev = lax.rem(qb * 4 + NC - 1, NC)
        qrows = qkv_ref[0, pl.ds(qb * QB, QB), :]
        krows = jnp.concatenate(
            [qkv_ref[0, pl.ds(cprev * BL, BL), :], qrows],
            axis=0)
        q = qrows[:, :DH]
        k = krows[:, :DH]
        nrm = jnp.sqrt(jnp.sum(k * k, axis=1, keepdims=True)) + 1e-6
        kn = k / nrm
        v2 = krows[:, DH:]
        scores = lax.dot_general(q, kn, (((1,), (1,)), ((), ())),
                                 precision=lax.Precision.DEFAULT,
                                 preferred_element_type=_f32)
        scores = scores * (1.0 / np.sqrt(DH)) + band_pen
        bq = sbq_ref[0, pl.ds(qb * QB, QB), :]
        pq = spq_ref[0, pl.ds(qb * QB, QB), :]
        bk = jnp.concatenate(
            [sbk_ref[0, pl.ds(cprev, 1), :]]
            + [sbk_ref[0, pl.ds(qb * 4 + j, 1), :] for j in range(4)],
            axis=1)
        pk = jnp.concatenate(
            [spk_ref[0, pl.ds(cprev, 1), :]]
            + [spk_ref[0, pl.ds(qb * 4 + j, 1), :] for j in range(4)],
            axis=1)
        scores = scores + jnp.where(bq == bk, 0.0, -1e9)
        scores = scores + jnp.where(pq == pk, -1e5, 0.0)
        m = jnp.max(scores, axis=1, keepdims=True)
        ex = jnp.exp(scores - m)
        sx = jnp.sum(ex, axis=1, keepdims=True)
        lse = m + jnp.log(sx)
        attn = ex / sx
        o = lax.dot_general(attn, v2, (((1,), (0,)), ((), ())),
                            precision=lax.Precision.DEFAULT,
                            preferred_element_type=_f32)
        packed = jnp.concatenate(
            [o, lse, jnp.zeros((QB, DH - 1), _f32)], axis=1)
        po_ref[0, pl.ds(qb * QB, QB), :] = packed
        return 0
    lax.fori_loop(0, NQB, block, 0)


def _attn(sqkv, sbq, sbk, spq, spk):
    return pl.pallas_call(
        _attn_body,
        grid=(NP,),
        in_specs=[
            pl.BlockSpec((1, S, 2 * DH), lambda p: (p, 0, 0)),
            pl.BlockSpec((1, S, 1), lambda p: (p, 0, 0)),
            pl.BlockSpec((1, NC, BL), lambda p: (p, 0, 0)),
            pl.BlockSpec((1, S, 1), lambda p: (p, 0, 0)),
            pl.BlockSpec((1, NC, BL), lambda p: (p, 0, 0)),
        ],
        out_specs=pl.BlockSpec((1, S, 2 * DH), lambda p: (p, 0, 0)),
        out_shape=jax.ShapeDtypeStruct((NP, S, 2 * DH), _f32),
    )(sqkv, sbq, sbk, spq, spk)


def _sc_unsort_body(uidx_hbm, of_hbm, ou_hbm, uidx_v, g_v, rb0, rb1,
                    sem0, sem1):
    p = lax.axis_index("s") * 2 + lax.axis_index("c")
    pltpu.sync_copy(uidx_hbm.at[p], uidx_v)

    def idx_body(i, _):
        g_v[pl.ds(i * 16, 16)] = uidx_v[pl.ds(i * 16, 16)] + p * S
        return 0
    lax.fori_loop(0, S // 16, idx_body, 0)

    bufs = ((rb0, sem0), (rb1, sem1))
    prev = None
    for c in range(S // 128):
        buf, sem = bufs[c % 2]
        cp = pltpu.async_copy(of_hbm.at[g_v.at[pl.ds(c * 128, 128)]],
                              buf, sem)
        if prev is not None:
            pcp, pbuf, pc = prev
            pcp.wait()
            pltpu.sync_copy(pbuf, ou_hbm.at[p, pl.ds(pc * 128, 128)])
        prev = (cp, buf, c)
    pcp, pbuf, pc = prev
    pcp.wait()
    pltpu.sync_copy(pbuf, ou_hbm.at[p, pl.ds(pc * 128, 128)])


@functools.cache
def _sc_unsort():
    mesh = plsc.VectorSubcoreMesh(core_axis_name="c", subcore_axis_name="s")
    return pl.kernel(
        _sc_unsort_body,
        out_type=[
            jax.ShapeDtypeStruct((NP, S, 2 * DH), _f32),
        ],
        mesh=mesh,
        scratch_types=[
            pltpu.VMEM((S,), _i32),
            pltpu.VMEM((S,), _i32),
            pltpu.VMEM((128, 2 * DH), _f32),
            pltpu.VMEM((128, 2 * DH), _f32),
            pltpu.SemaphoreType.DMA,
            pltpu.SemaphoreType.DMA,
        ],
        compiler_params=pltpu.CompilerParams(needs_layout_passes=False),
    )


def _post_body(o_ref, x1_ref, x2_ref, wo_ref, g_ref, b_ref,
               w1_ref, b1_ref, w2_ref, b2_ref, y1_ref, y2_ref):
    ou = o_ref[...]
    acc = x1_ref[...]
    for h in range(H):
        p0 = ou[h]
        p1 = ou[H + h]
        l0 = p0[:, DH:DH + 1]
        l1 = p1[:, DH:DH + 1]
        m = jnp.maximum(l0, l1)
        e0 = jnp.exp(l0 - m)
        e1 = jnp.exp(l1 - m)
        inv = 1.0 / (e0 + e1)
        comb = (e0 * inv) * p0[:, :DH] + (e1 * inv) * p1[:, :DH]
        acc = acc + _dot(comb, wo_ref[h * DH:(h + 1) * DH, :])
    y1_ref[...] = acc
    hn = _ln(acc, g_ref[...], b_ref[...])
    t = jnp.maximum(_dot(hn, w1_ref[...]) + b1_ref[...], 0.0)
    y2_ref[...] = x2_ref[...] + _dot(t, w2_ref[...]) + b2_ref[...]


def _post(o_u, x1, x2, Wo, g, b, W1, b1, W2, b2):
    return pl.pallas_call(
        _post_body,
        grid=(S // SB,),
        in_specs=[
            pl.BlockSpec((NP, SB, 2 * DH), lambda i: (0, i, 0)),
            pl.BlockSpec((SB, D), lambda i: (i, 0)),
            pl.BlockSpec((SB, D), lambda i: (i, 0)),
            pl.BlockSpec((D, D), lambda i: (0, 0)),
            pl.BlockSpec((1, D), lambda i: (0, 0)),
            pl.BlockSpec((1, D), lambda i: (0, 0)),
            pl.BlockSpec((D, F), lambda i: (0, 0)),
            pl.BlockSpec((1, F), lambda i: (0, 0)),
            pl.BlockSpec((F, D), lambda i: (0, 0)),
            pl.BlockSpec((1, D), lambda i: (0, 0)),
        ],
        out_specs=[
            pl.BlockSpec((SB, D), lambda i: (i, 0)),
            pl.BlockSpec((SB, D), lambda i: (i, 0)),
        ],
        out_shape=[
            jax.ShapeDtypeStruct((S, D), _f32),
            jax.ShapeDtypeStruct((S, D), _f32),
        ],
    )(o_u, x1, x2, Wo, g, b, W1, b1, W2, b2)


def _cls_body(x_ref, w1_ref, b1_ref, w2_ref, b2_ref, out_ref):
    t = jnp.maximum(_dot(x_ref[...], w1_ref[...]) + b1_ref[...], 0.0)
    out_ref[...] = _dot(t, w2_ref[...]) + b2_ref[...]


def _cls(x2, Wc1, bc1, Wc2p, bc2p):
    return pl.pallas_call(
        _cls_body,
        grid=(S // SB,),
        in_specs=[
            pl.BlockSpec((SB, D), lambda i: (i, 0)),
            pl.BlockSpec((D, 2 * D), lambda i: (0, 0)),
            pl.BlockSpec((1, 2 * D), lambda i: (0, 0)),
            pl.BlockSpec((2 * D, 128), lambda i: (0, 0)),
            pl.BlockSpec((1, 128), lambda i: (0, 0)),
        ],
        out_specs=pl.BlockSpec((SB, 128), lambda i: (i, 0)),
        out_shape=jax.ShapeDtypeStruct((S, 128), _f32),
    )(x2, Wc1, bc1, Wc2p, bc2p)


def kernel(inputs, Wqk, Wv, Wo, ln1_g, ln1_b, ln2_g, ln2_b, W1, b1, W2, b2,
           Wc1, bc1, Wc2, bc2):
    x0 = inputs.reshape(S, D)
    x1 = x0
    x2 = x0
    for li in range(NL):
        rkey = jax.random.fold_in(jax.random.key(123), li)
        R = jax.random.normal(rkey, (DH, ROUNDS, NB // 2), dtype=_f32)
        Wpack = jnp.concatenate(
            [Wqk[li].reshape(D, H, 1, DH), Wv[li].reshape(D, H, 1, DH)],
            axis=2).reshape(D, 2 * D)
        qkv, rot = _pre(x2, ln1_g[li].reshape(1, D), ln1_b[li].reshape(1, D),
                        Wpack, R.reshape(DH, ROUNDS * 16))
        uidx3, bkt3 = _route(rot)
        uidx = uidx3.reshape(NP, S)
        bkt = bkt3.reshape(NP, S)
        sqkv, sb, sp = _sc_sort_gather()(uidx, bkt,
                                         qkv.reshape(S * H, 2 * DH))
        sbf = sb.astype(_f32)
        spf = sp.astype(_f32)
        po = _attn(sqkv,
                   sbf.reshape(NP, S, 1), sbf.reshape(NP, NC, BL),
                   spf.reshape(NP, S, 1), spf.reshape(NP, NC, BL))
        (o_u,) = _sc_unsort()(uidx, po.reshape(NP * S, 2 * DH))
        y1, y2 = _post(o_u, x1, x2, Wo[li],
                       ln2_g[li].reshape(1, D), ln2_b[li].reshape(1, D),
                       W1[li], b1[li].reshape(1, F),
                       W2[li], b2[li].reshape(1, D))
        x1, x2 = y1, y2
    Wc2p = jnp.pad(Wc2, ((0, 0), (0, 128 - NCLS)))
    bc2p = jnp.pad(bc2, (0, 128 - NCLS)).reshape(1, 128)
    logits = _cls(x2, Wc1, bc1.reshape(1, 2 * D), Wc2p, bc2p)
    return logits[:, :NCLS].reshape(1, S, NCLS)


# speedup vs baseline: 1.1253x; 1.0001x over previous
import functools

import jax
import jax.numpy as jnp
import numpy as np
from jax import lax
from jax.experimental import pallas as pl
from jax.experimental.pallas import tpu as pltpu
from jax.experimental.pallas import tpu_sc as plsc

S = 2048
D = 1024
H = 16
DH = 64
F = 2048
NL = 2
ROUNDS = 2
BL = 64
NC = S // BL
NB = 32
NP = ROUNDS * H
NCLS = 10
SB = 256
HIGH = lax.Precision.HIGHEST

_f32 = jnp.float32
_i32 = jnp.int32


def _dot(a, b):
    return lax.dot_general(a, b, (((1,), (0,)), ((), ())),
                           precision=lax.Precision.DEFAULT,
                           preferred_element_type=_f32)


def _dot_hi(a, b):
    return lax.dot_general(a, b, (((1,), (0,)), ((), ())), precision=HIGH,
                           preferred_element_type=_f32)


def _ln(x, g, b):
    m = jnp.mean(x, -1, keepdims=True)
    v = jnp.mean((x - m) * (x - m), -1, keepdims=True)
    return g * (x - m) / jnp.sqrt(v + 1e-5) + b


def _pre_body(x_ref, g_ref, b_ref, wpack_ref, rcat_ref, out_ref, rot_ref):
    xn = _ln(x_ref[...], g_ref[...], b_ref[...])
    out = _dot(xn, wpack_ref[...])
    out_ref[...] = out
    rc = rcat_ref[...]
    for p in range(NP):
        h, r = p % H, p // H
        rot_ref[p] = _dot(out[:, h * 2 * DH:h * 2 * DH + DH],
                          rc[:, r * 16:r * 16 + 16])


def _pre(x2, g, b, Wpack, Rcat):
    return pl.pallas_call(
        _pre_body,
        grid=(S // SB,),
        in_specs=[
            pl.BlockSpec((SB, D), lambda i: (i, 0)),
            pl.BlockSpec((1, D), lambda i: (0, 0)),
            pl.BlockSpec((1, D), lambda i: (0, 0)),
            pl.BlockSpec((D, 2 * D), lambda i: (0, 0)),
            pl.BlockSpec((DH, ROUNDS * 16), lambda i: (0, 0)),
        ],
        out_specs=[
            pl.BlockSpec((SB, 2 * D), lambda i: (i, 0)),
            pl.BlockSpec((NP, SB, 16), lambda i: (0, i, 0)),
        ],
        out_shape=[
            jax.ShapeDtypeStruct((S, 2 * D), _f32),
            jax.ShapeDtypeStruct((NP, S, 16), _f32),
        ],
    )(x2, g, b, Wpack, Rcat)


def _route_body(rot_ref, uidx_ref, bkt_ref):
    io = lax.broadcasted_iota(_i32, (S, NB), 1)
    upper = (lax.broadcasted_iota(_i32, (NB, NB), 0)
             < lax.broadcasted_iota(_i32, (NB, NB), 1)).astype(_f32)
    ltri = (lax.broadcasted_iota(_i32, (SB, SB), 0)
            > lax.broadcasted_iota(_i32, (SB, SB), 1)).astype(_f32)
    rh = rot_ref[0]
    rc = jnp.concatenate([rh, -rh], axis=1)
    mx = jnp.max(rc, axis=1, keepdims=True)
    bkt = jnp.min(jnp.where(rc >= mx, io, 2 ** 30), axis=1, keepdims=True)
    oh = (bkt == io).astype(_f32)
    counts = jnp.sum(oh, axis=0, keepdims=True)
    offs = _dot_hi(counts, upper)
    running = jnp.zeros((1, NB), _f32)
    for i in range(S // SB):
        ohb = oh[i * SB:(i + 1) * SB]
        excl = _dot_hi(ltri, ohb) + running
        rank = jnp.sum(excl * ohb, axis=1, keepdims=True)
        base = jnp.sum(offs * ohb, axis=1, keepdims=True)
        uidx_ref[0, i * SB:(i + 1) * SB, :] = (rank + base).astype(_i32)
        running = running + jnp.sum(ohb, axis=0, keepdims=True)
    bkt_ref[0] = bkt


def _route(rot):
    return pl.pallas_call(
        _route_body,
        grid=(NP,),
        in_specs=[
            pl.BlockSpec((1, S, 16), lambda p: (p, 0, 0)),
        ],
        out_specs=[
            pl.BlockSpec((1, S, 1), lambda p: (p, 0, 0)),
            pl.BlockSpec((1, S, 1), lambda p: (p, 0, 0)),
        ],
        out_shape=[
            jax.ShapeDtypeStruct((NP, S, 1), _i32),
            jax.ShapeDtypeStruct((NP, S, 1), _i32),
        ],
    )(rot)


def _sc_sort_gather_body(uidx_hbm, bkt_hbm, qkv_hbm,
                         sqkv_hbm, sb_hbm, sp_hbm,
                         uidx_v, bkt_v, sidx_v, sb_v, g_v, rb0, rb1,
                         sem0, sem1):
    p = lax.axis_index("s") * 2 + lax.axis_index("c")
    h = lax.rem(p, H)
    pltpu.sync_copy(uidx_hbm.at[p], uidx_v)
    pltpu.sync_copy(bkt_hbm.at[p], bkt_v)

    def inv_body(i, _):
        idx = uidx_v[pl.ds(i * 16, 16)]
        plsc.store_scatter(sidx_v, [idx], lax.iota(_i32, 16) + i * 16)
        return 0
    lax.fori_loop(0, S // 16, inv_body, 0)

    bufs = ((rb0, sem0), (rb1, sem1))
    prev = None
    for c in range(S // 128):
        for j in range(8):
            i = c * 8 + j
            sv = sidx_v[pl.ds(i * 16, 16)]
            sb_v[pl.ds(i * 16, 16)] = plsc.load_gather(bkt_v, [sv])
            g_v[pl.ds(i * 16, 16)] = sv * H + h
        buf, sem = bufs[c % 2]
        cp = pltpu.async_copy(qkv_hbm.at[g_v.at[pl.ds(c * 128, 128)]],
                              buf, sem)
        if prev is not None:
            pcp, pbuf, pc = prev
            pcp.wait()
            pltpu.sync_copy(pbuf, sqkv_hbm.at[p, pl.ds(pc * 128, 128)])
        prev = (cp, buf, c)
    pcp, pbuf, pc = prev
    pcp.wait()
    pltpu.sync_copy(pbuf, sqkv_hbm.at[p, pl.ds(pc * 128, 128)])
    pltpu.sync_copy(sb_v, sb_hbm.at[p])
    pltpu.sync_copy(sidx_v, sp_hbm.at[p])


@functools.cache
def _sc_sort_gather():
    mesh = plsc.VectorSubcoreMesh(core_axis_name="c", subcore_axis_name="s")
    return pl.kernel(
        _sc_sort_gather_body,
        out_type=[
            jax.ShapeDtypeStruct((NP, S, 2 * DH), _f32),
            jax.ShapeDtypeStruct((NP, S), _i32),
            jax.ShapeDtypeStruct((NP, S), _i32),
        ],
        mesh=mesh,
        scratch_types=[
            pltpu.VMEM((S,), _i32),
            pltpu.VMEM((S,), _i32),
            pltpu.VMEM((S,), _i32),
            pltpu.VMEM((S,), _i32),
            pltpu.VMEM((S,), _i32),
            pltpu.VMEM((128, 2 * DH), _f32),
            pltpu.VMEM((128, 2 * DH), _f32),
            pltpu.SemaphoreType.DMA,
            pltpu.SemaphoreType.DMA,
        ],
        compiler_params=pltpu.CompilerParams(needs_layout_passes=False),
    )


QB = 256
KW = QB + BL
NQB = S // QB


def _attn_body(qkv_ref, sbq_ref, sbk_ref, spq_ref, spk_ref, po_ref):
    qci = lax.broadcasted_iota(_i32, (QB, KW), 0) // BL
    kci = lax.broadcasted_iota(_i32, (QB, KW), 1) // BL
    band_pen = jnp.where((kci == qci) | (kci == qci + 1), 0.0, -1e9)

    def block(qb, _):
        cprev = lax.rem(qb * 4 + NC - 1, NC)
        qrows = qkv_ref[0, pl.ds(qb * QB, QB), :]
        krows = jnp.concatenate(
            [qkv_ref[0, pl.ds(cprev * BL, BL), :], qrows],
            axis=0)
        q = qrows[:, :DH]
        k = krows[:, :DH]
        nrm = jnp.sqrt(jnp.sum(k * k, axis=1, keepdims=True)) + 1e-6
        kn = k / nrm
        v2 = krows[:, DH:]
        scores = lax.dot_general(q, kn, (((1,), (1,)), ((), ())),
                                 precision=lax.Precision.DEFAULT,
                                 preferred_element_type=_f32)
        scores = scores * (1.0 / np.sqrt(DH)) + band_pen
        bq = sbq_ref[0, pl.ds(qb * QB, QB), :]
        pq = spq_ref[0, pl.ds(qb * QB, QB), :]
        bk = jnp.concatenate(
            [sbk_ref[0, pl.ds(cprev, 1), :]]
            + [sbk_ref[0, pl.ds(qb * 4 + j, 1), :] for j in range(4)],
            axis=1)
        pk = jnp.concatenate(
            [spk_ref[0, pl.ds(cprev, 1), :]]
            + [spk_ref[0, pl.ds(qb * 4 + j, 1), :] for j in range(4)],
            axis=1)
        scores = scores + jnp.where(bq == bk, 0.0, -1e9)
        scores = scores + jnp.where(pq == pk, -1e5, 0.0)
        m = jnp.max(scores, axis=1, keepdims=True)
        ex = jnp.exp(scores - m)
        sx = jnp.sum(ex, axis=1, keepdims=True)
        lse = m + jnp.log(sx)
        attn = ex / sx
        o = lax.dot_general(attn, v2, (((1,), (0,)), ((), ())),
                            precision=lax.Precision.DEFAULT,
                            preferred_element_type=_f32)
        packed = jnp.concatenate(
            [o, lse, jnp.zeros((QB, DH - 1), _f32)], axis=1)
        po_ref[0, pl.ds(qb * QB, QB), :] = packed
        return 0
    lax.fori_loop(0, NQB, block, 0)


def _attn(sqkv, sbq, sbk, spq, spk):
    return pl.pallas_call(
        _attn_body,
        grid=(NP,),
        in_specs=[
            pl.BlockSpec((1, S, 2 * DH), lambda p: (p, 0, 0)),
            pl.BlockSpec((1, S, 1), lambda p: (p, 0, 0)),
            pl.BlockSpec((1, NC, BL), lambda p: (p, 0, 0)),
            pl.BlockSpec((1, S, 1), lambda p: (p, 0, 0)),
            pl.BlockSpec((1, NC, BL), lambda p: (p, 0, 0)),
        ],
        out_specs=pl.BlockSpec((1, S, 2 * DH), lambda p: (p, 0, 0)),
        out_shape=jax.ShapeDtypeStruct((NP, S, 2 * DH), _f32),
    )(sqkv, sbq, sbk, spq, spk)


def _sc_unsort_body(uidx_hbm, of_hbm, ou_hbm, uidx_v, g_v, rb0, rb1,
                    sem0, sem1):
    p = lax.axis_index("s") * 2 + lax.axis_index("c")
    pltpu.sync_copy(uidx_hbm.at[p], uidx_v)

    def idx_body(i, _):
        g_v[pl.ds(i * 16, 16)] = uidx_v[pl.ds(i * 16, 16)] + p * S
        return 0
    lax.fori_loop(0, S // 16, idx_body, 0)

    bufs = ((rb0, sem0), (rb1, sem1))
    prev = None
    for c in range(S // 128):
        buf, sem = bufs[c % 2]
        cp = pltpu.async_copy(of_hbm.at[g_v.at[pl.ds(c * 128, 128)]],
                              buf, sem)
        if prev is not None:
            pcp, pbuf, pc = prev
            pcp.wait()
            pltpu.sync_copy(pbuf, ou_hbm.at[p, pl.ds(pc * 128, 128)])
        prev = (cp, buf, c)
    pcp, pbuf, pc = prev
    pcp.wait()
    pltpu.sync_copy(pbuf, ou_hbm.at[p, pl.ds(pc * 128, 128)])


@functools.cache
def _sc_unsort():
    mesh = plsc.VectorSubcoreMesh(core_axis_name="c", subcore_axis_name="s")
    return pl.kernel(
        _sc_unsort_body,
        out_type=[
            jax.ShapeDtypeStruct((NP, S, 2 * DH), _f32),
        ],
        mesh=mesh,
        scratch_types=[
            pltpu.VMEM((S,), _i32),
            pltpu.VMEM((S,), _i32),
            pltpu.VMEM((128, 2 * DH), _f32),
            pltpu.VMEM((128, 2 * DH), _f32),
            pltpu.SemaphoreType.DMA,
            pltpu.SemaphoreType.DMA,
        ],
        compiler_params=pltpu.CompilerParams(needs_layout_passes=False),
    )


def _post_body(o_ref, x1_ref, x2_ref, wo_ref, g_ref, b_ref,
               w1_ref, b1_ref, w2_ref, b2_ref, y1_ref, y2_ref):
    ou = o_ref[...]
    acc = x1_ref[...]
    for h in range(H):
        p0 = ou[h]
        p1 = ou[H + h]
        l0 = p0[:, DH:DH + 1]
        l1 = p1[:, DH:DH + 1]
        m = jnp.maximum(l0, l1)
        e0 = jnp.exp(l0 - m)
        e1 = jnp.exp(l1 - m)
        inv = 1.0 / (e0 + e1)
        comb = (e0 * inv) * p0[:, :DH] + (e1 * inv) * p1[:, :DH]
        acc = acc + _dot(comb, wo_ref[h * DH:(h + 1) * DH, :])
    y1_ref[...] = acc
    hn = _ln(acc, g_ref[...], b_ref[...])
    t = jnp.maximum(_dot(hn, w1_ref[...]) + b1_ref[...], 0.0)
    y2_ref[...] = x2_ref[...] + _dot(t, w2_ref[...]) + b2_ref[...]


def _post(o_u, x1, x2, Wo, g, b, W1, b1, W2, b2):
    return pl.pallas_call(
        _post_body,
        grid=(S // SB,),
        in_specs=[
            pl.BlockSpec((NP, SB, 2 * DH), lambda i: (0, i, 0)),
            pl.BlockSpec((SB, D), lambda i: (i, 0)),
            pl.BlockSpec((SB, D), lambda i: (i, 0)),
            pl.BlockSpec((D, D), lambda i: (0, 0)),
            pl.BlockSpec((1, D), lambda i: (0, 0)),
            pl.BlockSpec((1, D), lambda i: (0, 0)),
            pl.BlockSpec((D, F), lambda i: (0, 0)),
            pl.BlockSpec((1, F), lambda i: (0, 0)),
            pl.BlockSpec((F, D), lambda i: (0, 0)),
            pl.BlockSpec((1, D), lambda i: (0, 0)),
        ],
        out_specs=[
            pl.BlockSpec((SB, D), lambda i: (i, 0)),
            pl.BlockSpec((SB, D), lambda i: (i, 0)),
        ],
        out_shape=[
            jax.ShapeDtypeStruct((S, D), _f32),
            jax.ShapeDtypeStruct((S, D), _f32),
        ],
    )(o_u, x1, x2, Wo, g, b, W1, b1, W2, b2)


def _post_cls_body(o_ref, x1_ref, x2_ref, wo_ref, g_ref, b_ref,
                   w1_ref, b1_ref, w2_ref, b2_ref,
                   wc1_ref, bc1_ref, wc2_ref, bc2_ref, out_ref):
    ou = o_ref[...]
    acc = x1_ref[...]
    for h in range(H):
        p0 = ou[h]
        p1 = ou[H + h]
        l0 = p0[:, DH:DH + 1]
        l1 = p1[:, DH:DH + 1]
        m = jnp.maximum(l0, l1)
        e0 = jnp.exp(l0 - m)
        e1 = jnp.exp(l1 - m)
        inv = 1.0 / (e0 + e1)
        comb = (e0 * inv) * p0[:, :DH] + (e1 * inv) * p1[:, :DH]
        acc = acc + _dot(comb, wo_ref[h * DH:(h + 1) * DH, :])
    hn = _ln(acc, g_ref[...], b_ref[...])
    t = jnp.maximum(_dot(hn, w1_ref[...]) + b1_ref[...], 0.0)
    y2 = x2_ref[...] + _dot(t, w2_ref[...]) + b2_ref[...]
    t2 = jnp.maximum(_dot(y2, wc1_ref[...]) + bc1_ref[...], 0.0)
    out_ref[...] = _dot(t2, wc2_ref[...]) + bc2_ref[...]


def _post_cls(o_u, x1, x2, Wo, g, b, W1, b1, W2, b2, Wc1, bc1, Wc2p, bc2p):
    return pl.pallas_call(
        _post_cls_body,
        grid=(S // SB,),
        in_specs=[
            pl.BlockSpec((NP, SB, 2 * DH), lambda i: (0, i, 0)),
            pl.BlockSpec((SB, D), lambda i: (i, 0)),
            pl.BlockSpec((SB, D), lambda i: (i, 0)),
            pl.BlockSpec((D, D), lambda i: (0, 0)),
            pl.BlockSpec((1, D), lambda i: (0, 0)),
            pl.BlockSpec((1, D), lambda i: (0, 0)),
            pl.BlockSpec((D, F), lambda i: (0, 0)),
            pl.BlockSpec((1, F), lambda i: (0, 0)),
            pl.BlockSpec((F, D), lambda i: (0, 0)),
            pl.BlockSpec((1, D), lambda i: (0, 0)),
            pl.BlockSpec((D, 2 * D), lambda i: (0, 0)),
            pl.BlockSpec((1, 2 * D), lambda i: (0, 0)),
            pl.BlockSpec((2 * D, 128), lambda i: (0, 0)),
            pl.BlockSpec((1, 128), lambda i: (0, 0)),
        ],
        out_specs=pl.BlockSpec((SB, 128), lambda i: (i, 0)),
        out_shape=jax.ShapeDtypeStruct((S, 128), _f32),
    )(o_u, x1, x2, Wo, g, b, W1, b1, W2, b2, Wc1, bc1, Wc2p, bc2p)


def kernel(inputs, Wqk, Wv, Wo, ln1_g, ln1_b, ln2_g, ln2_b, W1, b1, W2, b2,
           Wc1, bc1, Wc2, bc2):
    x0 = inputs.reshape(S, D)
    x1 = x0
    x2 = x0
    for li in range(NL):
        rkey = jax.random.fold_in(jax.random.key(123), li)
        R = jax.random.normal(rkey, (DH, ROUNDS, NB // 2), dtype=_f32)
        Wpack = jnp.concatenate(
            [Wqk[li].reshape(D, H, 1, DH), Wv[li].reshape(D, H, 1, DH)],
            axis=2).reshape(D, 2 * D)
        qkv, rot = _pre(x2, ln1_g[li].reshape(1, D), ln1_b[li].reshape(1, D),
                        Wpack, R.reshape(DH, ROUNDS * 16))
        uidx3, bkt3 = _route(rot)
        uidx = uidx3.reshape(NP, S)
        bkt = bkt3.reshape(NP, S)
        sqkv, sb, sp = _sc_sort_gather()(uidx, bkt,
                                         qkv.reshape(S * H, 2 * DH))
        sbf = sb.astype(_f32)
        spf = sp.astype(_f32)
        po = _attn(sqkv,
                   sbf.reshape(NP, S, 1), sbf.reshape(NP, NC, BL),
                   spf.reshape(NP, S, 1), spf.reshape(NP, NC, BL))
        (o_u,) = _sc_unsort()(uidx, po.reshape(NP * S, 2 * DH))
        if li < NL - 1:
            x1, x2 = _post(o_u, x1, x2, Wo[li],
                           ln2_g[li].reshape(1, D), ln2_b[li].reshape(1, D),
                           W1[li], b1[li].reshape(1, F),
                           W2[li], b2[li].reshape(1, D))
        else:
            Wc2p = jnp.pad(Wc2, ((0, 0), (0, 128 - NCLS)))
            bc2p = jnp.pad(bc2, (0, 128 - NCLS)).reshape(1, 128)
            logits = _post_cls(o_u, x1, x2, Wo[li],
                               ln2_g[li].reshape(1, D),
                               ln2_b[li].reshape(1, D),
                               W1[li], b1[li].reshape(1, F),
                               W2[li], b2[li].reshape(1, D),
                               Wc1, bc1.reshape(1, 2 * D), Wc2p, bc2p)
    return logits[:, :NCLS].reshape(1, S, NCLS)
